# Initial kernel scaffold; baseline (speedup 1.0000x reference)
import jax
import jax.numpy as jnp
from jax import lax
from jax.experimental import pallas as pl
from jax.experimental.pallas import tpu as pltpu
from jax.experimental.pallas import tpu_sc as plsc

NA = 50000
DEG = 16
D = 32
B = 16384
K = 20
REG = 0.01

NC, NS = 2, 16
NW = NC * NS
NPAD = 50176
NPW = NPAD // NW
CN = 56
CE = CN * DEG
NCH = NPW // CN
NSTR = CE // 128

RB = 6272
NBLK = NPAD // RB

_sc_mesh = plsc.VectorSubcoreMesh(
    core_axis_name="c", subcore_axis_name="s", num_cores=NC, num_subcores=NS)


def _prep_body(fa, fb, W, bb, a1, a2, *outs):
    ts = outs[:8]
    s2s = outs[8:16]
    s1s = outs[16:24]
    dst_is_b = (True, True, False, False, False, False, True, True)
    src_is_b = (False, False, False, False, True, True, True, True)
    fav = fa[...]
    fbv = fb[...]
    for a in range(8):
        fd = fbv if dst_is_b[a] else fav
        fs = fbv if src_is_b[a] else fav
        ne = jnp.dot(fd, W[a], preferred_element_type=jnp.float32) + bb[a]
        ts[a][...] = ne
        s2s[a][...] = jnp.dot(ne, a2[a], preferred_element_type=jnp.float32)
        s1s[a][...] = jnp.dot(fs, a1[a], preferred_element_type=jnp.float32)


def _prep(fa_p, fb_p, agg_W, agg_b, a1, a2):
    rb = pl.BlockSpec((RB, D), lambda i: (i, 0))
    cb = pl.BlockSpec((RB, 1), lambda i: (i, 0))
    full3 = pl.BlockSpec((8, D, D), lambda i: (0, 0, 0))
    full2 = pl.BlockSpec((8, D), lambda i: (0, 0))
    fulla = pl.BlockSpec((8, D, 1), lambda i: (0, 0, 0))
    out_shapes = ([jax.ShapeDtypeStruct((NPAD, D), jnp.float32)] * 8
                  + [jax.ShapeDtypeStruct((NPAD, 1), jnp.float32)] * 16)
    return pl.pallas_call(
        _prep_body,
        grid=(NBLK,),
        in_specs=[rb, rb, full3, full2, fulla, fulla],
        out_specs=[rb] * 8 + [cb] * 16,
        out_shape=out_shapes,
    )(fa_p, fb_p, agg_W, agg_b, a1, a2)


def _agg_body(*refs):
    tabs = refs[0:8]
    s2h = refs[8:16]
    s1h = refs[16:24]
    dsth = refs[24:32]
    outs = refs[32:40]
    s2_loc, s1_loc, idxb, rowb, outb, sem0, sem1 = refs[40:]
    sems = (sem0, sem1)

    wid = lax.axis_index("s") * NC + lax.axis_index("c")
    nb0 = wid * NPW
    eb0 = nb0 * DEG

    for a in range(8):
        T, S2, S1, DSTl, M = tabs[a], s2h[a], s1h[a], dsth[a], outs[a]
        pltpu.sync_copy(S2, s2_loc)
        pltpu.sync_copy(S1.at[pl.ds(nb0, NPW)], s1_loc)

        def fire(c, b):
            pltpu.sync_copy(DSTl.at[pl.ds(eb0 + c * CE, CE)], idxb.at[b])
            for j in range(NSTR):
                pltpu.async_copy(
                    T.at[idxb.at[b].at[pl.ds(j * 128, 128)]],
                    rowb.at[b].at[pl.ds(j * 128, 128)], sems[b])

        def drain(b):
            for j in range(NSTR):
                pltpu.make_async_copy(
                    T.at[idxb.at[b].at[pl.ds(j * 128, 128)]],
                    rowb.at[b].at[pl.ds(j * 128, 128)], sems[b]).wait()

        fire(0, 0)
        fire(1, 1)

        @pl.loop(0, NCH, step=2)
        def _chunks(c0):
            for b in range(2):
                c = c0 + b
                drain(b)

                @pl.loop(0, CN, unroll=2)
                def _node(nn):
                    e0 = nn * DEG
                    idxv = idxb[b, pl.ds(e0, DEG)]
                    s2g = plsc.load_gather(s2_loc, [idxv])
                    nloc = jnp.full((16,), 0, jnp.int32) + (c * CN + nn)
                    s1b = plsc.load_gather(s1_loc, [nloc])
                    t = s1b + s2g
                    val = jnp.exp(jnp.where(t > 0, t, 0.1 * (jnp.exp(t) - 1.0)))
                    rs = jnp.sum(val)
                    inv = 1.0 / jnp.where(rs == 0.0, 1.0, rs)
                    acc0 = jnp.zeros((16,), jnp.float32)
                    acc1 = jnp.zeros((16,), jnp.float32)
                    for k in range(DEG):
                        bk = val[k]
                        acc0 = acc0 + bk * rowb[b, e0 + k, 0:16]
                        acc1 = acc1 + bk * rowb[b, e0 + k, 16:32]
                    outb[nn, 0:16] = acc0 * inv
                    outb[nn, 16:32] = acc1 * inv

                pltpu.sync_copy(outb, M.at[pl.ds(nb0 + c * CN, CN)])

                @pl.when(c + 2 < NCH)
                def _():
                    fire(c + 2, b)


def _agg(tabs, s2s, s1s, dsts):
    out_type = [jax.ShapeDtypeStruct((NPAD, D), jnp.float32)] * 8
    scratch = [
        pltpu.VMEM((NPAD,), jnp.float32),
        pltpu.VMEM((NPW,), jnp.float32),
        pltpu.VMEM((2, CE), jnp.int32),
        pltpu.VMEM((2, CE, D), jnp.float32),
        pltpu.VMEM((CN, D), jnp.float32),
        pltpu.SemaphoreType.DMA,
        pltpu.SemaphoreType.DMA,
    ]
    return pl.kernel(
        _agg_body, out_type=out_type, mesh=_sc_mesh, scratch_types=scratch,
    )(*tabs, *s2s, *s1s, *dsts)


def _update_body(f, m0, m1, m2, m3, W1, b1, pa, W2, b2, o):
    x = jnp.concatenate([f[...], m0[...], m1[...], m2[...], m3[...]], axis=1)
    h = jnp.dot(x, W1[...], preferred_element_type=jnp.float32) + b1[...]
    h = jnp.where(h > 0, h, pa[0, 0] * h)
    o[...] = jnp.dot(h, W2[...], preferred_element_type=jnp.float32) + b2[...]


def _update(f_p, ms, up_W1, up_b1, pa2, up_W2, up_b2):
    rb = pl.BlockSpec((RB, D), lambda i: (i, 0))
    return pl.pallas_call(
        _update_body,
        grid=(NBLK,),
        in_specs=[rb] * 5 + [
            pl.BlockSpec((5 * D, 2 * D), lambda i: (0, 0)),
            pl.BlockSpec((1, 2 * D), lambda i: (0, 0)),
            pl.BlockSpec((1, 1), lambda i: (0, 0)),
            pl.BlockSpec((2 * D, D), lambda i: (0, 0)),
            pl.BlockSpec((1, D), lambda i: (0, 0)),
        ],
        out_specs=rb,
        out_shape=jax.ShapeDtypeStruct((NPAD, D), jnp.float32),
    )(f_p, *ms, up_W1, up_b1, pa2, up_W2, up_b2)


UVN_TOT = B * (K + 2)
GPW = UVN_TOT // NW
CG = 1024
GCH = GPW // CG
GSTR = CG // 128


def _gath_body(emb, uvn, out, idxg, rowg, sem0, sem1):
    sems = (sem0, sem1)
    wid = lax.axis_index("s") * NC + lax.axis_index("c")
    base = wid * GPW

    def fire(c, b):
        pltpu.sync_copy(uvn.at[pl.ds(base + c * CG, CG)], idxg.at[b])
        for j in range(GSTR):
            pltpu.async_copy(
                emb.at[idxg.at[b].at[pl.ds(j * 128, 128)]],
                rowg.at[b].at[pl.ds(j * 128, 128)], sems[b])

    def drain(b):
        for j in range(GSTR):
            pltpu.make_async_copy(
                emb.at[idxg.at[b].at[pl.ds(j * 128, 128)]],
                rowg.at[b].at[pl.ds(j * 128, 128)], sems[b]).wait()

    fire(0, 0)
    fire(1, 1)

    @pl.loop(0, GCH + 1, step=2)
    def _chunks(c0):
        for b in range(2):
            c = c0 + b

            @pl.when(c < GCH)
            def _():
                drain(b)
                pltpu.sync_copy(rowg.at[b], out.at[pl.ds(base + c * CG, CG)])

                @pl.when(c + 2 < GCH)
                def _():
                    fire(c + 2, b)


def _gather_rows(emb2, uvn):
    scratch = [
        pltpu.VMEM((2, CG), jnp.int32),
        pltpu.VMEM((2, CG, D), jnp.float32),
        pltpu.SemaphoreType.DMA,
        pltpu.SemaphoreType.DMA,
    ]
    return pl.kernel(
        _gath_body,
        out_type=jax.ShapeDtypeStruct((UVN_TOT, D), jnp.float32),
        mesh=_sc_mesh, scratch_types=scratch,
    )(emb2, uvn)


LB = 2048
NLB = B // LB


def _loss_body(en, eu, ev, w, o):
    i = pl.program_id(0)
    euv = eu[...]
    evv = ev[...]
    env = en[...].reshape(LB, K, D)
    wv = w[...]
    pos = jnp.sum(euv * evv, axis=1)
    neg = jnp.sum(euv[:, None, :] * env, axis=2)
    x = jnp.sign(wv) * (K * pos[:, None] - neg)
    ls = jnp.minimum(x, 0.0) - jnp.log(1.0 + jnp.exp(-jnp.abs(x)))
    part = -jnp.sum(ls) + REG * (jnp.sum(euv * euv) + jnp.sum(evv * evv)
                                 + jnp.sum(env * env))

    @pl.when(i == 0)
    def _():
        o[0, 0] = 0.0

    o[0, 0] += part


def _loss(euvn, w2):
    return pl.pallas_call(
        _loss_body,
        grid=(NLB,),
        in_specs=[
            pl.BlockSpec((LB * K, D), lambda i: (i, 0)),
            pl.BlockSpec((LB, D), lambda i: (B * K // LB + i, 0)),
            pl.BlockSpec((LB, D), lambda i: (B * K // LB + NLB + i, 0)),
            pl.BlockSpec((LB, 1), lambda i: (i, 0)),
        ],
        out_specs=pl.BlockSpec(memory_space=pltpu.SMEM),
        out_shape=jax.ShapeDtypeStruct((1, 1), jnp.float32),
    )(euvn, euvn, euvn, w2)


def kernel(e_ab_p, e_ab_n, e_ba_p, e_ba_n, e_aa_p, e_aa_n, e_bb_p, e_bb_n,
           feat_a, feat_b, agg_W, agg_b, agg_a,
           up_W1, up_b1, prelu_a, up_W2, up_b2,
           u, v, w, n):
    pad_n = ((0, NPAD - NA), (0, 0))
    fa_p = jnp.pad(feat_a, pad_n)
    fb_p = jnp.pad(feat_b, pad_n)
    a1 = agg_a[:, :D, :]
    a2 = agg_a[:, D:, :]

    edges = (e_ab_p, e_ab_n, e_aa_p, e_aa_n, e_ba_p, e_ba_n, e_bb_p, e_bb_n)
    dsts = tuple(
        jnp.pad(e[:, 1].astype(jnp.int32), (0, (NPAD - NA) * DEG))
        for e in edges)

    prep = _prep(fa_p, fb_p, agg_W, agg_b, a1, a2)
    tabs = prep[0:8]
    s2s = tuple(x.reshape(NPAD) for x in prep[8:16])
    s1s = tuple(x.reshape(NPAD) for x in prep[16:24])

    ms = _agg(tabs, s2s, s1s, dsts)

    pa2 = prelu_a.reshape(1, 1)
    b1r = up_b1.reshape(1, 2 * D)
    b2r = up_b2.reshape(1, D)
    new_a = _update(fa_p, ms[0:4], up_W1, b1r, pa2, up_W2, b2r)
    new_b = _update(fb_p, ms[4:8], up_W1, b1r, pa2, up_W2, b2r)
    emb2 = jnp.concatenate([new_a, new_b], axis=0)

    shift = jnp.int32(NPAD - NA)
    remap = lambda i: (i + jnp.where(i >= NA, shift, 0)).astype(jnp.int32)
    uvn = jnp.concatenate([remap(n.reshape(-1)), remap(u), remap(v)])

    euvn = _gather_rows(emb2, uvn)
    res = _loss(euvn, w.reshape(B, 1))
    return res[0, 0]


# speedup vs baseline: 20.3206x; 20.3206x over previous
import jax
import jax.numpy as jnp
from jax import lax
from jax.experimental import pallas as pl
from jax.experimental.pallas import tpu as pltpu
from jax.experimental.pallas import tpu_sc as plsc

NA = 50000
DEG = 16
D = 32
B = 16384
K = 20
REG = 0.01

NC, NS = 2, 16
NW = NC * NS
NPAD = 50176
NPW = NPAD // NW
CN = 56
CE = CN * DEG
NCH = NPW // CN
NSTR = CE // 128

RB = 1568
NBLK = NPAD // RB

_SC_PARAMS = pltpu.CompilerParams(
    needs_layout_passes=False, use_tc_tiling_on_sc=False)

_sc_mesh = plsc.VectorSubcoreMesh(
    core_axis_name="c", subcore_axis_name="s", num_cores=NC, num_subcores=NS)


def _prep_body(fa, fb, W, bb, a1, a2, *outs):
    ts = outs[:8]
    s2c, s1c = outs[8], outs[9]
    dst_is_b = (True, True, False, False, False, False, True, True)
    src_is_b = (False, False, False, False, True, True, True, True)
    fav = fa[...]
    fbv = fb[...]
    s2l, s1l = [], []
    for a in range(8):
        fd = fbv if dst_is_b[a] else fav
        fs = fbv if src_is_b[a] else fav
        ne = jnp.dot(fd, W[a], preferred_element_type=jnp.float32) + bb[a]
        ts[a][...] = ne
        s2l.append(jnp.dot(ne, a2[a], preferred_element_type=jnp.float32))
        s1l.append(jnp.dot(fs, a1[a], preferred_element_type=jnp.float32))
    s2c[...] = jnp.concatenate(s2l, axis=1)
    s1c[...] = jnp.concatenate(s1l, axis=1)


def _prep(fa_p, fb_p, agg_W, agg_b, a1, a2):
    rb = pl.BlockSpec((RB, D), lambda i: (i, 0))
    cb = pl.BlockSpec((RB, 8), lambda i: (i, 0))
    full3 = pl.BlockSpec((8, D, D), lambda i: (0, 0, 0))
    full2 = pl.BlockSpec((8, D), lambda i: (0, 0))
    fulla = pl.BlockSpec((8, D, 1), lambda i: (0, 0, 0))
    out_shapes = ([jax.ShapeDtypeStruct((NPAD, D), jnp.float32)] * 8
                  + [jax.ShapeDtypeStruct((NPAD, 8), jnp.float32)] * 2)
    return pl.pallas_call(
        _prep_body,
        grid=(NBLK,),
        in_specs=[rb, rb, full3, full2, fulla, fulla],
        out_specs=[rb] * 8 + [cb] * 2,
        out_shape=out_shapes,
    )(fa_p, fb_p, agg_W, agg_b, a1, a2)


def _agg_body(*refs):
    tabs = refs[0:8]
    s2r = refs[8]
    s1r = refs[9]
    dsth = refs[10:18]
    outs = refs[18:26]
    s2_loc, s1_loc, idxb, rowb, outb, sem0, sem1 = refs[26:]
    sems = (sem0, sem1)

    wid = lax.axis_index("s") * NC + lax.axis_index("c")
    nb0 = wid * NPW
    eb0 = nb0 * DEG

    for a in range(8):
        T, DSTl, M = tabs[a], dsth[a], outs[a]
        pltpu.sync_copy(s2r.at[a], s2_loc)
        pltpu.sync_copy(s1r.at[a].at[pl.ds(nb0, NPW)], s1_loc)

        def fire(c, b):
            pltpu.sync_copy(DSTl.at[pl.ds(eb0 + c * CE, CE)], idxb.at[b])
            for j in range(NSTR):
                pltpu.async_copy(
                    T.at[idxb.at[b].at[pl.ds(j * 128, 128)]],
                    rowb.at[b].at[pl.ds(j * 128, 128)], sems[b])

        def drain(b):
            for j in range(NSTR):
                pltpu.make_async_copy(
                    T.at[idxb.at[b].at[pl.ds(j * 128, 128)]],
                    rowb.at[b].at[pl.ds(j * 128, 128)], sems[b]).wait()

        fire(0, 0)
        fire(1, 1)

        @pl.loop(0, NCH, step=2)
        def _chunks(c0):
            for b in range(2):
                c = c0 + b
                drain(b)

                @pl.loop(0, CN, unroll=2)
                def _node(nn):
                    e0 = nn * DEG
                    idxv = idxb[b, pl.ds(e0, DEG)]
                    s2g = plsc.load_gather(s2_loc, [idxv])
                    nloc = jnp.full((16,), 0, jnp.int32) + (c * CN + nn)
                    s1b = plsc.load_gather(s1_loc, [nloc])
                    t = s1b + s2g
                    val = jnp.exp(jnp.where(t > 0, t, 0.1 * (jnp.exp(t) - 1.0)))
                    rs = jnp.sum(val)
                    rsv = jnp.where(rs == 0.0, 1.0, rs) + jnp.zeros(
                        (16,), jnp.float32)
                    inv = jnp.ones((16,), jnp.float32) / rsv
                    acc0 = jnp.zeros((16,), jnp.float32)
                    acc1 = jnp.zeros((16,), jnp.float32)
                    for k in range(DEG):
                        bk = val[k]
                        acc0 = acc0 + bk * rowb[b, e0 + k, 0:16]
                        acc1 = acc1 + bk * rowb[b, e0 + k, 16:32]
                    outb[nn, 0:16] = acc0 * inv
                    outb[nn, 16:32] = acc1 * inv

                pltpu.sync_copy(outb, M.at[pl.ds(nb0 + c * CN, CN)])

                @pl.when(c + 2 < NCH)
                def _():
                    fire(c + 2, b)


def _agg(tabs, s2r, s1r, dsts):
    out_type = [jax.ShapeDtypeStruct((NPAD, D), jnp.float32)] * 8
    scratch = [
        pltpu.VMEM((NPAD,), jnp.float32),
        pltpu.VMEM((NPW,), jnp.float32),
        pltpu.VMEM((2, CE), jnp.int32),
        pltpu.VMEM((2, CE, D), jnp.float32),
        pltpu.VMEM((CN, D), jnp.float32),
        pltpu.SemaphoreType.DMA,
        pltpu.SemaphoreType.DMA,
    ]
    return pl.kernel(
        _agg_body, out_type=out_type, mesh=_sc_mesh, scratch_types=scratch,
        compiler_params=_SC_PARAMS,
    )(*tabs, s2r, s1r, *dsts)


def _update_body(f, m0, m1, m2, m3, W1, b1, pa, W2, b2, o):
    x = jnp.concatenate([f[...], m0[...], m1[...], m2[...], m3[...]], axis=1)
    h = jnp.dot(x, W1[...], preferred_element_type=jnp.float32) + b1[...]
    h = jnp.where(h > 0, h, pa[0, 0] * h)
    o[...] = jnp.dot(h, W2[...], preferred_element_type=jnp.float32) + b2[...]


def _update(f_p, ms, up_W1, up_b1, pa2, up_W2, up_b2):
    rb = pl.BlockSpec((RB, D), lambda i: (i, 0))
    return pl.pallas_call(
        _update_body,
        grid=(NBLK,),
        in_specs=[rb] * 5 + [
            pl.BlockSpec((5 * D, 2 * D), lambda i: (0, 0)),
            pl.BlockSpec((1, 2 * D), lambda i: (0, 0)),
            pl.BlockSpec((1, 1), lambda i: (0, 0)),
            pl.BlockSpec((2 * D, D), lambda i: (0, 0)),
            pl.BlockSpec((1, D), lambda i: (0, 0)),
        ],
        out_specs=rb,
        out_shape=jax.ShapeDtypeStruct((NPAD, D), jnp.float32),
    )(f_p, *ms, up_W1, up_b1, pa2, up_W2, up_b2)


UVN_TOT = B * (K + 2)
GPW = UVN_TOT // NW
CG = 1024
GCH = GPW // CG
GSTR = CG // 128


def _gath_body(emb, uvn, out, idxg, rowg, sem0, sem1):
    sems = (sem0, sem1)
    wid = lax.axis_index("s") * NC + lax.axis_index("c")
    base = wid * GPW

    def fire(c, b):
        pltpu.sync_copy(uvn.at[pl.ds(base + c * CG, CG)], idxg.at[b])
        for j in range(GSTR):
            pltpu.async_copy(
                emb.at[idxg.at[b].at[pl.ds(j * 128, 128)]],
                rowg.at[b].at[pl.ds(j * 128, 128)], sems[b])

    def drain(b):
        for j in range(GSTR):
            pltpu.make_async_copy(
                emb.at[idxg.at[b].at[pl.ds(j * 128, 128)]],
                rowg.at[b].at[pl.ds(j * 128, 128)], sems[b]).wait()

    fire(0, 0)
    fire(1, 1)

    @pl.loop(0, GCH + 1, step=2)
    def _chunks(c0):
        for b in range(2):
            c = c0 + b

            @pl.when(c < GCH)
            def _():
                drain(b)
                pltpu.sync_copy(rowg.at[b], out.at[pl.ds(base + c * CG, CG)])

                @pl.when(c + 2 < GCH)
                def _():
                    fire(c + 2, b)


def _gather_rows(emb2, uvn):
    scratch = [
        pltpu.VMEM((2, CG), jnp.int32),
        pltpu.VMEM((2, CG, D), jnp.float32),
        pltpu.SemaphoreType.DMA,
        pltpu.SemaphoreType.DMA,
    ]
    return pl.kernel(
        _gath_body,
        out_type=jax.ShapeDtypeStruct((UVN_TOT, D), jnp.float32),
        mesh=_sc_mesh, scratch_types=scratch,
        compiler_params=_SC_PARAMS,
    )(emb2, uvn)


LB = 2048
NLB = B // LB


def _loss_body(en, eu, ev, w, o):
    i = pl.program_id(0)
    euv = eu[...]
    evv = ev[...]
    env = en[...].reshape(LB, K, D)
    wv = w[...]
    pos = jnp.sum(euv * evv, axis=1)
    neg = jnp.sum(euv[:, None, :] * env, axis=2)
    x = jnp.sign(wv) * (K * pos[:, None] - neg)
    ls = jnp.minimum(x, 0.0) - jnp.log(1.0 + jnp.exp(-jnp.abs(x)))
    part = -jnp.sum(ls) + REG * (jnp.sum(euv * euv) + jnp.sum(evv * evv)
                                 + jnp.sum(env * env))

    @pl.when(i == 0)
    def _():
        o[0, 0] = 0.0

    o[0, 0] += part


def _loss(euvn, w2):
    return pl.pallas_call(
        _loss_body,
        grid=(NLB,),
        in_specs=[
            pl.BlockSpec((LB * K, D), lambda i: (i, 0)),
            pl.BlockSpec((LB, D), lambda i: (B * K // LB + i, 0)),
            pl.BlockSpec((LB, D), lambda i: (B * K // LB + NLB + i, 0)),
            pl.BlockSpec((LB, 1), lambda i: (i, 0)),
        ],
        out_specs=pl.BlockSpec(memory_space=pltpu.SMEM),
        out_shape=jax.ShapeDtypeStruct((1, 1), jnp.float32),
    )(euvn, euvn, euvn, w2)


def kernel(e_ab_p, e_ab_n, e_ba_p, e_ba_n, e_aa_p, e_aa_n, e_bb_p, e_bb_n,
           feat_a, feat_b, agg_W, agg_b, agg_a,
           up_W1, up_b1, prelu_a, up_W2, up_b2,
           u, v, w, n):
    pad_n = ((0, NPAD - NA), (0, 0))
    fa_p = jnp.pad(feat_a, pad_n)
    fb_p = jnp.pad(feat_b, pad_n)
    a1 = agg_a[:, :D, :]
    a2 = agg_a[:, D:, :]

    edges = (e_ab_p, e_ab_n, e_aa_p, e_aa_n, e_ba_p, e_ba_n, e_bb_p, e_bb_n)
    dsts = tuple(
        jnp.pad(e[:, 1].astype(jnp.int32), (0, (NPAD - NA) * DEG))
        for e in edges)

    prep = _prep(fa_p, fb_p, agg_W, agg_b, a1, a2)
    tabs = prep[0:8]
    s2r = prep[8].T
    s1r = prep[9].T

    ms = _agg(tabs, s2r, s1r, dsts)

    pa2 = prelu_a.reshape(1, 1)
    b1r = up_b1.reshape(1, 2 * D)
    b2r = up_b2.reshape(1, D)
    new_a = _update(fa_p, ms[0:4], up_W1, b1r, pa2, up_W2, b2r)
    new_b = _update(fb_p, ms[4:8], up_W1, b1r, pa2, up_W2, b2r)
    emb2 = jnp.concatenate([new_a, new_b], axis=0)

    shift = jnp.int32(NPAD - NA)
    remap = lambda i: (i + jnp.where(i >= NA, shift, 0)).astype(jnp.int32)
    uvn = jnp.concatenate([remap(n.reshape(-1)), remap(u), remap(v)])

    euvn = _gather_rows(emb2, uvn)
    res = _loss(euvn, w.reshape(B, 1))
    return res[0, 0]
